# Initial kernel scaffold; baseline (speedup 1.0000x reference)
import jax
import jax.numpy as jnp
from jax.experimental import pallas as pl


def kernel(edge_index, x, W1, b1, W2, b2, mW1, mb1, mW2, mb2):
    raise NotImplementedError("write your pallas kernel here")


# speedup vs baseline: 13.6702x; 13.6702x over previous
import functools

import jax
import jax.numpy as jnp
from jax import lax
from jax.experimental import pallas as pl
from jax.experimental.pallas import tpu as pltpu
from jax.experimental.pallas import tpu_sc as plsc

N = 10000
E = 320000
D = 128
H = 32
C = 2

NC = 2
NS = 16
NW = NC * NS
EW = E // NW
CH = 80
NCH = EW // CH
NB = 5
NG = NCH // NB
EV = 624
TAIL = N - EV * NS

_mesh = plsc.VectorSubcoreMesh(core_axis_name="c", subcore_axis_name="s")


@functools.partial(
    pl.kernel,
    out_type=(
        jax.ShapeDtypeStruct((NC, N, 16), jnp.float32),
        jax.ShapeDtypeStruct((NC, N, 16), jnp.float32),
    ),
    mesh=_mesh,
    compiler_params=pltpu.CompilerParams(use_tc_tiling_on_sc=False),
    scratch_types=[
        pltpu.VMEM((NCH, CH), jnp.int32),
        pltpu.VMEM((NCH, CH), jnp.int32),
        pltpu.VMEM((CH, 16), jnp.float32),
        pltpu.VMEM_SHARED((N, 16), jnp.float32),
        pltpu.VMEM_SHARED((N, 16), jnp.float32),
    ],
)
def _deg_kernel(src_hbm, dst_hbm, ones_hbm, z16_hbm,
                dout_hbm, din_hbm, sidx, didx, ones, acc_o, acc_i):
    c = lax.axis_index("c")
    s = lax.axis_index("s")
    wid = c * NS + s
    pltpu.sync_copy(src_hbm.at[wid], sidx)
    pltpu.sync_copy(dst_hbm.at[wid], didx)
    pltpu.sync_copy(ones_hbm, ones)
    r0 = s * EV
    pltpu.sync_copy(z16_hbm, acc_o.at[pl.ds(r0, EV)])
    pltpu.sync_copy(z16_hbm, acc_i.at[pl.ds(r0, EV)])

    @pl.when(s == NS - 1)
    def _zero_tail():
        pltpu.sync_copy(z16_hbm.at[pl.ds(0, TAIL)], acc_o.at[pl.ds(N - TAIL, TAIL)])
        pltpu.sync_copy(z16_hbm.at[pl.ds(0, TAIL)], acc_i.at[pl.ds(N - TAIL, TAIL)])

    plsc.subcore_barrier()

    def chunk(j, carry):
        pltpu.sync_copy(ones, acc_o.at[sidx.at[j]], add=True)
        pltpu.sync_copy(ones, acc_i.at[didx.at[j]], add=True)
        return carry

    lax.fori_loop(0, NCH, chunk, 0)
    plsc.subcore_barrier()
    pltpu.sync_copy(acc_o.at[pl.ds(r0, EV)], dout_hbm.at[c, pl.ds(r0, EV)])
    pltpu.sync_copy(acc_i.at[pl.ds(r0, EV)], din_hbm.at[c, pl.ds(r0, EV)])

    @pl.when(s == NS - 1)
    def _evac_tail():
        pltpu.sync_copy(acc_o.at[pl.ds(N - TAIL, TAIL)], dout_hbm.at[c, pl.ds(N - TAIL, TAIL)])
        pltpu.sync_copy(acc_i.at[pl.ds(N - TAIL, TAIL)], din_hbm.at[c, pl.ds(N - TAIL, TAIL)])


@functools.partial(
    pl.kernel,
    out_type=jax.ShapeDtypeStruct((NC, N, H), jnp.float32),
    mesh=_mesh,
    compiler_params=pltpu.CompilerParams(use_tc_tiling_on_sc=False),
    scratch_types=[
        pltpu.VMEM((NCH, CH), jnp.int32),
        pltpu.VMEM((NCH, CH), jnp.int32),
        pltpu.VMEM((NB, CH, H), jnp.float32),
        pltpu.VMEM_SHARED((N, H), jnp.float32),
        pltpu.SemaphoreType.DMA,
        pltpu.SemaphoreType.DMA,
        pltpu.SemaphoreType.DMA,
        pltpu.SemaphoreType.DMA,
        pltpu.SemaphoreType.DMA,
    ],
)
def _agg_kernel(h_hbm, src_hbm, dst_hbm, z32_hbm, out_hbm,
                sidx, didx, rows, acc, sem0, sem1, sem2, sem3, sem4):
    sems = (sem0, sem1, sem2, sem3, sem4)
    c = lax.axis_index("c")
    s = lax.axis_index("s")
    wid = c * NS + s
    pltpu.sync_copy(src_hbm.at[wid], sidx)
    pltpu.sync_copy(dst_hbm.at[wid], didx)
    for b in range(NB):
        pltpu.async_copy(h_hbm.at[sidx.at[b]], rows.at[b], sems[b])
    r0 = s * EV
    pltpu.sync_copy(z32_hbm, acc.at[pl.ds(r0, EV)])

    @pl.when(s == NS - 1)
    def _zero_tail():
        pltpu.sync_copy(z32_hbm.at[pl.ds(0, TAIL)], acc.at[pl.ds(N - TAIL, TAIL)])

    plsc.subcore_barrier()

    def group(g, carry):
        base = g * NB
        for b in range(NB):
            j = base + b
            pltpu.make_async_copy(h_hbm.at[sidx.at[j]], rows.at[b], sems[b]).wait()
            pltpu.sync_copy(rows.at[b], acc.at[didx.at[j]], add=True)

            @pl.when(j + NB < NCH)
            def _issue():
                pltpu.async_copy(h_hbm.at[sidx.at[j + NB]], rows.at[b], sems[b])

        return carry

    lax.fori_loop(0, NG, group, 0)
    plsc.subcore_barrier()
    pltpu.sync_copy(acc.at[pl.ds(r0, EV)], out_hbm.at[c, pl.ds(r0, EV)])

    @pl.when(s == NS - 1)
    def _evac_tail():
        pltpu.sync_copy(acc.at[pl.ds(N - TAIL, TAIL)], out_hbm.at[c, pl.ds(N - TAIL, TAIL)])


RB = 1000
_GRID = N // RB


def _mm1_body(x_ref, w_ref, o_ref):
    o_ref[...] = jnp.dot(x_ref[...], w_ref[...], preferred_element_type=jnp.float32)


def _norm1_body(do_ref, di_ref, p_ref, h_ref, doi_ref, dii_ref):
    deg_o = do_ref[0] + do_ref[1]
    deg_i = di_ref[0] + di_ref[1]
    doi = lax.rsqrt(jnp.maximum(deg_o, 1.0))
    dii = lax.rsqrt(jnp.maximum(deg_i, 1.0))
    doi_ref[...] = doi
    dii_ref[...] = dii
    h_ref[...] = p_ref[...] * doi[:, 0:1]


def _layer2_body(a_ref, dii_ref, doi_ref, b1_ref, w2_ref, o_ref):
    agg = a_ref[0] + a_ref[1]
    h1 = jnp.maximum(agg * dii_ref[...][:, 0:1] + b1_ref[...], 0.0)
    o_ref[...] = jnp.dot(h1, w2_ref[...],
                         preferred_element_type=jnp.float32) * doi_ref[...][:, 0:1]


def _head_body(a_ref, dii_ref, b2_ref, mw1_ref, mb1_ref, mw2_ref, mb2_ref, o_ref):
    agg = a_ref[0] + a_ref[1]
    h2 = jnp.maximum(agg * dii_ref[...][:, 0:1] + b2_ref[...], 0.0)
    m = jnp.maximum(
        jnp.dot(h2, mw1_ref[...], preferred_element_type=jnp.float32) + mb1_ref[...],
        0.0)
    o_ref[...] = jnp.dot(m, mw2_ref[...],
                         preferred_element_type=jnp.float32) + mb2_ref[...]


@jax.jit
def kernel(edge_index, x, W1, b1, W2, b2, mW1, mb1, mW2, mb2):
    ei = edge_index.reshape(2, NW, NCH, CH)
    src3d = ei[0]
    dst3d = ei[1]
    ones16 = jnp.ones((CH, 16), jnp.float32)
    z16 = jnp.zeros((EV, 16), jnp.float32)
    z32 = jnp.zeros((EV, H), jnp.float32)

    dout_p, din_p = _deg_kernel(src3d, dst3d, ones16, z16)

    p1 = pl.pallas_call(
        _mm1_body,
        grid=(_GRID,),
        in_specs=[
            pl.BlockSpec((RB, D), lambda i: (i, 0)),
            pl.BlockSpec((D, H), lambda i: (0, 0)),
        ],
        out_specs=pl.BlockSpec((RB, H), lambda i: (i, 0)),
        out_shape=jax.ShapeDtypeStruct((N, H), jnp.float32),
    )(x, W1)

    h1s, doi, dii = pl.pallas_call(
        _norm1_body,
        grid=(_GRID,),
        in_specs=[
            pl.BlockSpec((NC, RB, 16), lambda i: (0, i, 0)),
            pl.BlockSpec((NC, RB, 16), lambda i: (0, i, 0)),
            pl.BlockSpec((RB, H), lambda i: (i, 0)),
        ],
        out_specs=[
            pl.BlockSpec((RB, H), lambda i: (i, 0)),
            pl.BlockSpec((RB, 16), lambda i: (i, 0)),
            pl.BlockSpec((RB, 16), lambda i: (i, 0)),
        ],
        out_shape=[
            jax.ShapeDtypeStruct((N, H), jnp.float32),
            jax.ShapeDtypeStruct((N, 16), jnp.float32),
            jax.ShapeDtypeStruct((N, 16), jnp.float32),
        ],
    )(dout_p, din_p, p1)

    a1 = _agg_kernel(h1s, src3d, dst3d, z32)

    h2s = pl.pallas_call(
        _layer2_body,
        grid=(_GRID,),
        in_specs=[
            pl.BlockSpec((NC, RB, H), lambda i: (0, i, 0)),
            pl.BlockSpec((RB, 16), lambda i: (i, 0)),
            pl.BlockSpec((RB, 16), lambda i: (i, 0)),
            pl.BlockSpec((1, H), lambda i: (0, 0)),
            pl.BlockSpec((H, H), lambda i: (0, 0)),
        ],
        out_specs=pl.BlockSpec((RB, H), lambda i: (i, 0)),
        out_shape=jax.ShapeDtypeStruct((N, H), jnp.float32),
    )(a1, dii, doi, b1.reshape(1, H), W2)

    a2 = _agg_kernel(h2s, src3d, dst3d, z32)

    out = pl.pallas_call(
        _head_body,
        grid=(_GRID,),
        in_specs=[
            pl.BlockSpec((NC, RB, H), lambda i: (0, i, 0)),
            pl.BlockSpec((RB, 16), lambda i: (i, 0)),
            pl.BlockSpec((1, H), lambda i: (0, 0)),
            pl.BlockSpec((H, H), lambda i: (0, 0)),
            pl.BlockSpec((1, H), lambda i: (0, 0)),
            pl.BlockSpec((H, C), lambda i: (0, 0)),
            pl.BlockSpec((1, C), lambda i: (0, 0)),
        ],
        out_specs=pl.BlockSpec((RB, C), lambda i: (i, 0)),
        out_shape=jax.ShapeDtypeStruct((N, C), jnp.float32),
    )(a2, dii, b2.reshape(1, H), mW1, mb1.reshape(1, H), mW2, mb2.reshape(1, C))

    return out


# speedup vs baseline: 14.8020x; 1.0828x over previous
import functools

import jax
import jax.numpy as jnp
from jax import lax
from jax.experimental import pallas as pl
from jax.experimental.pallas import tpu as pltpu
from jax.experimental.pallas import tpu_sc as plsc

N = 10000
E = 320000
D = 128
H = 32
C = 2

NC = 2
NS = 16
NW = NC * NS
EW = E // NW
CH = 80
NCH = EW // CH
NBUF = 10
LEAD = 5
NGRP = (NCH - LEAD) // NBUF
EV = 624
TAIL = N - EV * NS

_mesh = plsc.VectorSubcoreMesh(core_axis_name="c", subcore_axis_name="s")


@functools.partial(
    pl.kernel,
    out_type=(
        jax.ShapeDtypeStruct((NC, N, 16), jnp.float32),
        jax.ShapeDtypeStruct((NC, N, 16), jnp.float32),
    ),
    mesh=_mesh,
    compiler_params=pltpu.CompilerParams(use_tc_tiling_on_sc=False),
    scratch_types=[
        pltpu.VMEM((NCH, CH), jnp.int32),
        pltpu.VMEM((NCH, CH), jnp.int32),
        pltpu.VMEM((CH, 16), jnp.float32),
        pltpu.VMEM_SHARED((N, 16), jnp.float32),
        pltpu.VMEM_SHARED((N, 16), jnp.float32),
        pltpu.SemaphoreType.DMA,
    ],
)
def _deg_kernel(src_hbm, dst_hbm, ones_hbm, z16_hbm,
                dout_hbm, din_hbm, sidx, didx, ones, acc_o, acc_i, sem0):
    c = lax.axis_index("c")
    s = lax.axis_index("s")
    wid = c * NS + s
    pltpu.sync_copy(src_hbm.at[wid], sidx)
    pltpu.sync_copy(dst_hbm.at[wid], didx)
    pltpu.sync_copy(ones_hbm, ones)
    r0 = s * EV
    pltpu.sync_copy(z16_hbm, acc_o.at[pl.ds(r0, EV)])
    pltpu.sync_copy(z16_hbm, acc_i.at[pl.ds(r0, EV)])

    @pl.when(s == NS - 1)
    def _zero_tail():
        pltpu.sync_copy(z16_hbm.at[pl.ds(0, TAIL)], acc_o.at[pl.ds(N - TAIL, TAIL)])
        pltpu.sync_copy(z16_hbm.at[pl.ds(0, TAIL)], acc_i.at[pl.ds(N - TAIL, TAIL)])

    plsc.subcore_barrier()

    LAG = 10

    def chunk(j, carry):
        pltpu.async_copy(ones, acc_o.at[sidx.at[j]], sem0, add=True)
        pltpu.async_copy(ones, acc_i.at[didx.at[j]], sem0, add=True)

        @pl.when(j >= LAG)
        def _drain_lagged():
            pltpu.make_async_copy(ones, acc_o.at[sidx.at[j - LAG]], sem0).wait()
            pltpu.make_async_copy(ones, acc_i.at[didx.at[j - LAG]], sem0).wait()

        return carry

    lax.fori_loop(0, NCH, chunk, 0)

    def drain(j, carry):
        pltpu.make_async_copy(ones, acc_o.at[sidx.at[j]], sem0).wait()
        pltpu.make_async_copy(ones, acc_i.at[didx.at[j]], sem0).wait()
        return carry

    lax.fori_loop(NCH - LAG, NCH, drain, 0)
    plsc.subcore_barrier()
    pltpu.sync_copy(acc_o.at[pl.ds(r0, EV)], dout_hbm.at[c, pl.ds(r0, EV)])
    pltpu.sync_copy(acc_i.at[pl.ds(r0, EV)], din_hbm.at[c, pl.ds(r0, EV)])

    @pl.when(s == NS - 1)
    def _evac_tail():
        pltpu.sync_copy(acc_o.at[pl.ds(N - TAIL, TAIL)], dout_hbm.at[c, pl.ds(N - TAIL, TAIL)])
        pltpu.sync_copy(acc_i.at[pl.ds(N - TAIL, TAIL)], din_hbm.at[c, pl.ds(N - TAIL, TAIL)])


@functools.partial(
    pl.kernel,
    out_type=jax.ShapeDtypeStruct((NC, N, H), jnp.float32),
    mesh=_mesh,
    compiler_params=pltpu.CompilerParams(use_tc_tiling_on_sc=False),
    scratch_types=[
        pltpu.VMEM((NCH, CH), jnp.int32),
        pltpu.VMEM((NCH, CH), jnp.int32),
        pltpu.VMEM((NBUF, CH, H), jnp.float32),
        pltpu.VMEM_SHARED((N, H), jnp.float32),
        pltpu.SemaphoreType.DMA((NBUF,)),
        pltpu.SemaphoreType.DMA((NBUF,)),
    ],
)
def _agg_kernel(h_hbm, src_hbm, dst_hbm, z32_hbm, out_hbm,
                sidx, didx, rows, acc, gsem, ssem):
    c = lax.axis_index("c")
    s = lax.axis_index("s")
    wid = c * NS + s
    pltpu.sync_copy(src_hbm.at[wid], sidx)
    pltpu.sync_copy(dst_hbm.at[wid], didx)
    for b in range(LEAD):
        pltpu.async_copy(h_hbm.at[sidx.at[b]], rows.at[b], gsem.at[b])
    r0 = s * EV
    pltpu.sync_copy(z32_hbm, acc.at[pl.ds(r0, EV)])

    @pl.when(s == NS - 1)
    def _zero_tail():
        pltpu.sync_copy(z32_hbm.at[pl.ds(0, TAIL)], acc.at[pl.ds(N - TAIL, TAIL)])

    plsc.subcore_barrier()

    def group(g, carry):
        base = g * NBUF
        for b in range(NBUF):
            j = base + b
            bw = (b - LEAD) % NBUF
            if b < LEAD:
                @pl.when(g >= 1)
                def _retire():
                    pltpu.make_async_copy(rows.at[bw], acc.at[didx.at[j - LEAD]],
                                          ssem.at[bw]).wait()
            else:
                pltpu.make_async_copy(rows.at[bw], acc.at[didx.at[j - LEAD]],
                                      ssem.at[bw]).wait()
            bg = (b + LEAD) % NBUF
            pltpu.async_copy(h_hbm.at[sidx.at[j + LEAD]], rows.at[bg], gsem.at[bg])
            pltpu.make_async_copy(h_hbm.at[sidx.at[j]], rows.at[b], gsem.at[b]).wait()
            pltpu.async_copy(rows.at[b], acc.at[didx.at[j]], ssem.at[b], add=True)
        return carry

    lax.fori_loop(0, NGRP, group, 0)

    for b in range(NCH - NGRP * NBUF):
        j = NGRP * NBUF + b
        bw = (b + NBUF - LEAD) % NBUF
        pltpu.make_async_copy(rows.at[bw], acc.at[didx.at[j - LEAD]], ssem.at[bw]).wait()
        pltpu.make_async_copy(h_hbm.at[sidx.at[j]], rows.at[b], gsem.at[b]).wait()
        pltpu.async_copy(rows.at[b], acc.at[didx.at[j]], ssem.at[b], add=True)
    for b in range(NCH - NGRP * NBUF):
        j = NGRP * NBUF + b
        pltpu.make_async_copy(rows.at[b], acc.at[didx.at[j]], ssem.at[b]).wait()

    plsc.subcore_barrier()
    pltpu.sync_copy(acc.at[pl.ds(r0, EV)], out_hbm.at[c, pl.ds(r0, EV)])

    @pl.when(s == NS - 1)
    def _evac_tail():
        pltpu.sync_copy(acc.at[pl.ds(N - TAIL, TAIL)], out_hbm.at[c, pl.ds(N - TAIL, TAIL)])


RB = 1000
_GRID = N // RB


def _mm1_body(x_ref, w_ref, o_ref):
    o_ref[...] = jnp.dot(x_ref[...], w_ref[...], preferred_element_type=jnp.float32)


def _norm1_body(do_ref, di_ref, p_ref, h_ref, doi_ref, dii_ref):
    deg_o = do_ref[0] + do_ref[1]
    deg_i = di_ref[0] + di_ref[1]
    doi = lax.rsqrt(jnp.maximum(deg_o, 1.0))
    dii = lax.rsqrt(jnp.maximum(deg_i, 1.0))
    doi_ref[...] = doi
    dii_ref[...] = dii
    h_ref[...] = p_ref[...] * doi[:, 0:1]


def _layer2_body(a_ref, dii_ref, doi_ref, b1_ref, w2_ref, o_ref):
    agg = a_ref[0] + a_ref[1]
    h1 = jnp.maximum(agg * dii_ref[...][:, 0:1] + b1_ref[...], 0.0)
    o_ref[...] = jnp.dot(h1, w2_ref[...],
                         preferred_element_type=jnp.float32) * doi_ref[...][:, 0:1]


def _head_body(a_ref, dii_ref, b2_ref, mw1_ref, mb1_ref, mw2_ref, mb2_ref, o_ref):
    agg = a_ref[0] + a_ref[1]
    h2 = jnp.maximum(agg * dii_ref[...][:, 0:1] + b2_ref[...], 0.0)
    m = jnp.maximum(
        jnp.dot(h2, mw1_ref[...], preferred_element_type=jnp.float32) + mb1_ref[...],
        0.0)
    o_ref[...] = jnp.dot(m, mw2_ref[...],
                         preferred_element_type=jnp.float32) + mb2_ref[...]


@jax.jit
def kernel(edge_index, x, W1, b1, W2, b2, mW1, mb1, mW2, mb2):
    ei = edge_index.reshape(2, NW, NCH, CH)
    src3d = ei[0]
    dst3d = ei[1]
    ones16 = jnp.ones((CH, 16), jnp.float32)
    z16 = jnp.zeros((EV, 16), jnp.float32)
    z32 = jnp.zeros((EV, H), jnp.float32)

    dout_p, din_p = _deg_kernel(src3d, dst3d, ones16, z16)

    p1 = pl.pallas_call(
        _mm1_body,
        grid=(_GRID,),
        in_specs=[
            pl.BlockSpec((RB, D), lambda i: (i, 0)),
            pl.BlockSpec((D, H), lambda i: (0, 0)),
        ],
        out_specs=pl.BlockSpec((RB, H), lambda i: (i, 0)),
        out_shape=jax.ShapeDtypeStruct((N, H), jnp.float32),
    )(x, W1)

    h1s, doi, dii = pl.pallas_call(
        _norm1_body,
        grid=(_GRID,),
        in_specs=[
            pl.BlockSpec((NC, RB, 16), lambda i: (0, i, 0)),
            pl.BlockSpec((NC, RB, 16), lambda i: (0, i, 0)),
            pl.BlockSpec((RB, H), lambda i: (i, 0)),
        ],
        out_specs=[
            pl.BlockSpec((RB, H), lambda i: (i, 0)),
            pl.BlockSpec((RB, 16), lambda i: (i, 0)),
            pl.BlockSpec((RB, 16), lambda i: (i, 0)),
        ],
        out_shape=[
            jax.ShapeDtypeStruct((N, H), jnp.float32),
            jax.ShapeDtypeStruct((N, 16), jnp.float32),
            jax.ShapeDtypeStruct((N, 16), jnp.float32),
        ],
    )(dout_p, din_p, p1)

    a1 = _agg_kernel(h1s, src3d, dst3d, z32)

    h2s = pl.pallas_call(
        _layer2_body,
        grid=(_GRID,),
        in_specs=[
            pl.BlockSpec((NC, RB, H), lambda i: (0, i, 0)),
            pl.BlockSpec((RB, 16), lambda i: (i, 0)),
            pl.BlockSpec((RB, 16), lambda i: (i, 0)),
            pl.BlockSpec((1, H), lambda i: (0, 0)),
            pl.BlockSpec((H, H), lambda i: (0, 0)),
        ],
        out_specs=pl.BlockSpec((RB, H), lambda i: (i, 0)),
        out_shape=jax.ShapeDtypeStruct((N, H), jnp.float32),
    )(a1, dii, doi, b1.reshape(1, H), W2)

    a2 = _agg_kernel(h2s, src3d, dst3d, z32)

    out = pl.pallas_call(
        _head_body,
        grid=(_GRID,),
        in_specs=[
            pl.BlockSpec((NC, RB, H), lambda i: (0, i, 0)),
            pl.BlockSpec((RB, 16), lambda i: (i, 0)),
            pl.BlockSpec((1, H), lambda i: (0, 0)),
            pl.BlockSpec((H, H), lambda i: (0, 0)),
            pl.BlockSpec((1, H), lambda i: (0, 0)),
            pl.BlockSpec((H, C), lambda i: (0, 0)),
            pl.BlockSpec((1, C), lambda i: (0, 0)),
        ],
        out_specs=pl.BlockSpec((RB, C), lambda i: (i, 0)),
        out_shape=jax.ShapeDtypeStruct((N, C), jnp.float32),
    )(a2, dii, b2.reshape(1, H), mW1, mb1.reshape(1, H), mW2, mb2.reshape(1, C))

    return out


# speedup vs baseline: 16.3169x; 1.1023x over previous
import functools

import jax
import jax.numpy as jnp
from jax import lax
from jax.experimental import pallas as pl
from jax.experimental.pallas import tpu as pltpu
from jax.experimental.pallas import tpu_sc as plsc

N = 10000
E = 320000
D = 128
H = 32
C = 2

NC = 2
NS = 16
NW = NC * NS
EW = E // NW
CH = 80
NCH = EW // CH
NBUF = 10
LEAD = 5
NGRP = (NCH - LEAD) // NBUF
EV = 624
TAIL = N - EV * NS

_mesh = plsc.VectorSubcoreMesh(core_axis_name="c", subcore_axis_name="s")


@functools.partial(
    pl.kernel,
    out_type=(
        jax.ShapeDtypeStruct((NC, N, 16), jnp.float32),
        jax.ShapeDtypeStruct((NC, N, 16), jnp.float32),
    ),
    mesh=_mesh,
    compiler_params=pltpu.CompilerParams(use_tc_tiling_on_sc=False),
    scratch_types=[
        pltpu.VMEM((NCH, CH), jnp.int32),
        pltpu.VMEM((NCH, CH), jnp.int32),
        pltpu.VMEM((CH, 16), jnp.float32),
        pltpu.VMEM_SHARED((N, 16), jnp.float32),
        pltpu.VMEM_SHARED((N, 16), jnp.float32),
        pltpu.SemaphoreType.DMA,
    ],
)
def _deg_kernel(ei_hbm, ones_hbm, z16_hbm,
                dout_hbm, din_hbm, sidx, didx, ones, acc_o, acc_i, sem0):
    c = lax.axis_index("c")
    s = lax.axis_index("s")
    wid = c * NS + s
    pltpu.sync_copy(ei_hbm.at[0, wid], sidx)
    pltpu.sync_copy(ei_hbm.at[1, wid], didx)
    pltpu.sync_copy(ones_hbm, ones)
    r0 = s * EV
    pltpu.sync_copy(z16_hbm, acc_o.at[pl.ds(r0, EV)])
    pltpu.sync_copy(z16_hbm, acc_i.at[pl.ds(r0, EV)])

    @pl.when(s == NS - 1)
    def _zero_tail():
        pltpu.sync_copy(z16_hbm.at[pl.ds(0, TAIL)], acc_o.at[pl.ds(N - TAIL, TAIL)])
        pltpu.sync_copy(z16_hbm.at[pl.ds(0, TAIL)], acc_i.at[pl.ds(N - TAIL, TAIL)])

    plsc.subcore_barrier()

    LAG = 10

    def chunk(j, carry):
        pltpu.async_copy(ones, acc_o.at[sidx.at[j]], sem0, add=True)
        pltpu.async_copy(ones, acc_i.at[didx.at[j]], sem0, add=True)

        @pl.when(j >= LAG)
        def _drain_lagged():
            pltpu.make_async_copy(ones, acc_o.at[sidx.at[j - LAG]], sem0).wait()
            pltpu.make_async_copy(ones, acc_i.at[didx.at[j - LAG]], sem0).wait()

        return carry

    lax.fori_loop(0, NCH, chunk, 0)

    def drain(j, carry):
        pltpu.make_async_copy(ones, acc_o.at[sidx.at[j]], sem0).wait()
        pltpu.make_async_copy(ones, acc_i.at[didx.at[j]], sem0).wait()
        return carry

    lax.fori_loop(NCH - LAG, NCH, drain, 0)
    plsc.subcore_barrier()
    pltpu.sync_copy(acc_o.at[pl.ds(r0, EV)], dout_hbm.at[c, pl.ds(r0, EV)])
    pltpu.sync_copy(acc_i.at[pl.ds(r0, EV)], din_hbm.at[c, pl.ds(r0, EV)])

    @pl.when(s == NS - 1)
    def _evac_tail():
        pltpu.sync_copy(acc_o.at[pl.ds(N - TAIL, TAIL)], dout_hbm.at[c, pl.ds(N - TAIL, TAIL)])
        pltpu.sync_copy(acc_i.at[pl.ds(N - TAIL, TAIL)], din_hbm.at[c, pl.ds(N - TAIL, TAIL)])


@functools.partial(
    pl.kernel,
    out_type=jax.ShapeDtypeStruct((NC, N, H), jnp.float32),
    mesh=_mesh,
    compiler_params=pltpu.CompilerParams(use_tc_tiling_on_sc=False),
    scratch_types=[
        pltpu.VMEM((NCH, CH), jnp.int32),
        pltpu.VMEM((NCH, CH), jnp.int32),
        pltpu.VMEM((NBUF, CH, H), jnp.float32),
        pltpu.VMEM_SHARED((N, H), jnp.float32),
        pltpu.SemaphoreType.DMA((NBUF,)),
        pltpu.SemaphoreType.DMA((NBUF,)),
    ],
)
def _agg_kernel(h_hbm, ei_hbm, z32_hbm, out_hbm,
                sidx, didx, rows, acc, gsem, ssem):
    c = lax.axis_index("c")
    s = lax.axis_index("s")
    wid = c * NS + s
    pltpu.sync_copy(ei_hbm.at[0, wid], sidx)
    pltpu.sync_copy(ei_hbm.at[1, wid], didx)
    for b in range(LEAD):
        pltpu.async_copy(h_hbm.at[sidx.at[b]], rows.at[b], gsem.at[b])
    r0 = s * EV
    pltpu.sync_copy(z32_hbm, acc.at[pl.ds(r0, EV)])

    @pl.when(s == NS - 1)
    def _zero_tail():
        pltpu.sync_copy(z32_hbm.at[pl.ds(0, TAIL)], acc.at[pl.ds(N - TAIL, TAIL)])

    plsc.subcore_barrier()

    def group(g, carry):
        base = g * NBUF
        for b in range(NBUF):
            j = base + b
            bw = (b - LEAD) % NBUF
            if b < LEAD:
                @pl.when(g >= 1)
                def _retire():
                    pltpu.make_async_copy(rows.at[bw], acc.at[didx.at[j - LEAD]],
                                          ssem.at[bw]).wait()
            else:
                pltpu.make_async_copy(rows.at[bw], acc.at[didx.at[j - LEAD]],
                                      ssem.at[bw]).wait()
            bg = (b + LEAD) % NBUF
            pltpu.async_copy(h_hbm.at[sidx.at[j + LEAD]], rows.at[bg], gsem.at[bg])
            pltpu.make_async_copy(h_hbm.at[sidx.at[j]], rows.at[b], gsem.at[b]).wait()
            pltpu.async_copy(rows.at[b], acc.at[didx.at[j]], ssem.at[b], add=True)
        return carry

    lax.fori_loop(0, NGRP, group, 0)

    for b in range(NCH - NGRP * NBUF):
        j = NGRP * NBUF + b
        bw = (b + NBUF - LEAD) % NBUF
        pltpu.make_async_copy(rows.at[bw], acc.at[didx.at[j - LEAD]], ssem.at[bw]).wait()
        pltpu.make_async_copy(h_hbm.at[sidx.at[j]], rows.at[b], gsem.at[b]).wait()
        pltpu.async_copy(rows.at[b], acc.at[didx.at[j]], ssem.at[b], add=True)
    for b in range(NCH - NGRP * NBUF):
        j = NGRP * NBUF + b
        pltpu.make_async_copy(rows.at[b], acc.at[didx.at[j]], ssem.at[b]).wait()

    plsc.subcore_barrier()
    pltpu.sync_copy(acc.at[pl.ds(r0, EV)], out_hbm.at[c, pl.ds(r0, EV)])

    @pl.when(s == NS - 1)
    def _evac_tail():
        pltpu.sync_copy(acc.at[pl.ds(N - TAIL, TAIL)], out_hbm.at[c, pl.ds(N - TAIL, TAIL)])


RB = 1000
_GRID = N // RB


def _mm1_body(x_ref, w_ref, o_ref):
    o_ref[...] = jnp.dot(x_ref[...], w_ref[...], preferred_element_type=jnp.float32)


def _norm1_body(do_ref, di_ref, p_ref, h_ref, doi_ref, dii_ref):
    deg_o = do_ref[0] + do_ref[1]
    deg_i = di_ref[0] + di_ref[1]
    doi = lax.rsqrt(jnp.maximum(deg_o, 1.0))
    dii = lax.rsqrt(jnp.maximum(deg_i, 1.0))
    doi_ref[...] = doi
    dii_ref[...] = dii
    h_ref[...] = p_ref[...] * doi[:, 0:1]


def _layer2_body(a_ref, dii_ref, doi_ref, b1_ref, w2_ref, o_ref):
    agg = a_ref[0] + a_ref[1]
    h1 = jnp.maximum(agg * dii_ref[...][:, 0:1] + b1_ref[...], 0.0)
    o_ref[...] = jnp.dot(h1, w2_ref[...],
                         preferred_element_type=jnp.float32) * doi_ref[...][:, 0:1]


def _head_body(a_ref, dii_ref, b2_ref, mw1_ref, mb1_ref, mw2_ref, mb2_ref, o_ref):
    agg = a_ref[0] + a_ref[1]
    h2 = jnp.maximum(agg * dii_ref[...][:, 0:1] + b2_ref[...], 0.0)
    m = jnp.maximum(
        jnp.dot(h2, mw1_ref[...], preferred_element_type=jnp.float32) + mb1_ref[...],
        0.0)
    o_ref[...] = jnp.dot(m, mw2_ref[...],
                         preferred_element_type=jnp.float32) + mb2_ref[...]


@jax.jit
def kernel(edge_index, x, W1, b1, W2, b2, mW1, mb1, mW2, mb2):
    ei = edge_index.reshape(2, NW, NCH, CH)
    ones16 = jnp.ones((CH, 16), jnp.float32)
    z16 = jnp.zeros((EV, 16), jnp.float32)
    z32 = jnp.zeros((EV, H), jnp.float32)

    dout_p, din_p = _deg_kernel(ei, ones16, z16)

    p1 = pl.pallas_call(
        _mm1_body,
        grid=(_GRID,),
        in_specs=[
            pl.BlockSpec((RB, D), lambda i: (i, 0)),
            pl.BlockSpec((D, H), lambda i: (0, 0)),
        ],
        out_specs=pl.BlockSpec((RB, H), lambda i: (i, 0)),
        out_shape=jax.ShapeDtypeStruct((N, H), jnp.float32),
    )(x, W1)

    h1s, doi, dii = pl.pallas_call(
        _norm1_body,
        grid=(_GRID,),
        in_specs=[
            pl.BlockSpec((NC, RB, 16), lambda i: (0, i, 0)),
            pl.BlockSpec((NC, RB, 16), lambda i: (0, i, 0)),
            pl.BlockSpec((RB, H), lambda i: (i, 0)),
        ],
        out_specs=[
            pl.BlockSpec((RB, H), lambda i: (i, 0)),
            pl.BlockSpec((RB, 16), lambda i: (i, 0)),
            pl.BlockSpec((RB, 16), lambda i: (i, 0)),
        ],
        out_shape=[
            jax.ShapeDtypeStruct((N, H), jnp.float32),
            jax.ShapeDtypeStruct((N, 16), jnp.float32),
            jax.ShapeDtypeStruct((N, 16), jnp.float32),
        ],
    )(dout_p, din_p, p1)

    a1 = _agg_kernel(h1s, ei, z32)

    h2s = pl.pallas_call(
        _layer2_body,
        grid=(_GRID,),
        in_specs=[
            pl.BlockSpec((NC, RB, H), lambda i: (0, i, 0)),
            pl.BlockSpec((RB, 16), lambda i: (i, 0)),
            pl.BlockSpec((RB, 16), lambda i: (i, 0)),
            pl.BlockSpec((1, H), lambda i: (0, 0)),
            pl.BlockSpec((H, H), lambda i: (0, 0)),
        ],
        out_specs=pl.BlockSpec((RB, H), lambda i: (i, 0)),
        out_shape=jax.ShapeDtypeStruct((N, H), jnp.float32),
    )(a1, dii, doi, b1.reshape(1, H), W2)

    a2 = _agg_kernel(h2s, ei, z32)

    out = pl.pallas_call(
        _head_body,
        grid=(_GRID,),
        in_specs=[
            pl.BlockSpec((NC, RB, H), lambda i: (0, i, 0)),
            pl.BlockSpec((RB, 16), lambda i: (i, 0)),
            pl.BlockSpec((1, H), lambda i: (0, 0)),
            pl.BlockSpec((H, H), lambda i: (0, 0)),
            pl.BlockSpec((1, H), lambda i: (0, 0)),
            pl.BlockSpec((H, C), lambda i: (0, 0)),
            pl.BlockSpec((1, C), lambda i: (0, 0)),
        ],
        out_specs=pl.BlockSpec((RB, C), lambda i: (i, 0)),
        out_shape=jax.ShapeDtypeStruct((N, C), jnp.float32),
    )(a2, dii, b2.reshape(1, H), mW1, mb1.reshape(1, H), mW2, mb2.reshape(1, C))

    return out


# speedup vs baseline: 17.0513x; 1.0450x over previous
import functools

import jax
import jax.numpy as jnp
from jax import lax
from jax.experimental import pallas as pl
from jax.experimental.pallas import tpu as pltpu
from jax.experimental.pallas import tpu_sc as plsc

N = 10000
E = 320000
D = 128
H = 32
C = 2

NC = 2
NS = 16
NW = NC * NS
EW = E // NW
CH = 80
NCH = EW // CH
NBUF = 10
LEAD = 5
NGRP = (NCH - LEAD) // NBUF
EV = 624
TAIL = N - EV * NS

_mesh = plsc.VectorSubcoreMesh(core_axis_name="c", subcore_axis_name="s")


@functools.partial(
    pl.kernel,
    out_type=(
        jax.ShapeDtypeStruct((NC, N, 16), jnp.float32),
        jax.ShapeDtypeStruct((NC, N, 16), jnp.float32),
    ),
    mesh=_mesh,
    compiler_params=pltpu.CompilerParams(use_tc_tiling_on_sc=False),
    scratch_types=[
        pltpu.VMEM((NCH, CH), jnp.int32),
        pltpu.VMEM((NCH, CH), jnp.int32),
        pltpu.VMEM((CH, 16), jnp.float32),
        pltpu.VMEM_SHARED((N, 16), jnp.float32),
        pltpu.VMEM_SHARED((N, 16), jnp.float32),
        pltpu.SemaphoreType.DMA,
    ],
)
def _deg_kernel(ei_hbm, ones_hbm, z16_hbm,
                dout_hbm, din_hbm, sidx, didx, ones, acc_o, acc_i, sem0):
    c = lax.axis_index("c")
    s = lax.axis_index("s")
    wid = c * NS + s
    pltpu.sync_copy(ei_hbm.at[0, wid], sidx)
    pltpu.sync_copy(ei_hbm.at[1, wid], didx)
    pltpu.sync_copy(ones_hbm, ones)
    r0 = s * EV
    pltpu.sync_copy(z16_hbm, acc_o.at[pl.ds(r0, EV)])
    pltpu.sync_copy(z16_hbm, acc_i.at[pl.ds(r0, EV)])

    @pl.when(s == NS - 1)
    def _zero_tail():
        pltpu.sync_copy(z16_hbm.at[pl.ds(0, TAIL)], acc_o.at[pl.ds(N - TAIL, TAIL)])
        pltpu.sync_copy(z16_hbm.at[pl.ds(0, TAIL)], acc_i.at[pl.ds(N - TAIL, TAIL)])

    plsc.subcore_barrier()

    LAG = 10

    def chunk(j, carry):
        pltpu.async_copy(ones, acc_o.at[sidx.at[j]], sem0, add=True)
        pltpu.async_copy(ones, acc_i.at[didx.at[j]], sem0, add=True)

        @pl.when(j >= LAG)
        def _drain_lagged():
            pltpu.make_async_copy(ones, acc_o.at[sidx.at[j - LAG]], sem0).wait()
            pltpu.make_async_copy(ones, acc_i.at[didx.at[j - LAG]], sem0).wait()

        return carry

    lax.fori_loop(0, NCH, chunk, 0)

    def drain(j, carry):
        pltpu.make_async_copy(ones, acc_o.at[sidx.at[j]], sem0).wait()
        pltpu.make_async_copy(ones, acc_i.at[didx.at[j]], sem0).wait()
        return carry

    lax.fori_loop(NCH - LAG, NCH, drain, 0)
    plsc.subcore_barrier()
    pltpu.sync_copy(acc_o.at[pl.ds(r0, EV)], dout_hbm.at[c, pl.ds(r0, EV)])
    pltpu.sync_copy(acc_i.at[pl.ds(r0, EV)], din_hbm.at[c, pl.ds(r0, EV)])

    @pl.when(s == NS - 1)
    def _evac_tail():
        pltpu.sync_copy(acc_o.at[pl.ds(N - TAIL, TAIL)], dout_hbm.at[c, pl.ds(N - TAIL, TAIL)])
        pltpu.sync_copy(acc_i.at[pl.ds(N - TAIL, TAIL)], din_hbm.at[c, pl.ds(N - TAIL, TAIL)])


@functools.partial(
    pl.kernel,
    out_type=jax.ShapeDtypeStruct((NC, N, H), jnp.float32),
    mesh=_mesh,
    compiler_params=pltpu.CompilerParams(use_tc_tiling_on_sc=False),
    scratch_types=[
        pltpu.VMEM((NCH, CH), jnp.int32),
        pltpu.VMEM((NCH, CH), jnp.int32),
        pltpu.VMEM((NBUF, CH, H), jnp.float32),
        pltpu.VMEM_SHARED((N, H), jnp.float32),
        pltpu.SemaphoreType.DMA((NBUF,)),
        pltpu.SemaphoreType.DMA((NBUF,)),
    ],
)
def _agg_kernel(h_hbm, ei_hbm, z32_hbm, out_hbm,
                sidx, didx, rows, acc, gsem, ssem):
    c = lax.axis_index("c")
    s = lax.axis_index("s")
    wid = c * NS + s
    pltpu.sync_copy(ei_hbm.at[0, wid], sidx)
    pltpu.sync_copy(ei_hbm.at[1, wid], didx)
    for b in range(LEAD):
        pltpu.async_copy(h_hbm.at[sidx.at[b]], rows.at[b], gsem.at[b])
    r0 = s * EV
    pltpu.sync_copy(z32_hbm, acc.at[pl.ds(r0, EV)])

    @pl.when(s == NS - 1)
    def _zero_tail():
        pltpu.sync_copy(z32_hbm.at[pl.ds(0, TAIL)], acc.at[pl.ds(N - TAIL, TAIL)])

    plsc.subcore_barrier()

    def group(g, carry):
        base = g * NBUF
        for b in range(NBUF):
            j = base + b
            bw = (b - LEAD) % NBUF
            if b < LEAD:
                @pl.when(g >= 1)
                def _retire():
                    pltpu.make_async_copy(rows.at[bw], acc.at[didx.at[j - LEAD]],
                                          ssem.at[bw]).wait()
            else:
                pltpu.make_async_copy(rows.at[bw], acc.at[didx.at[j - LEAD]],
                                      ssem.at[bw]).wait()
            bg = (b + LEAD) % NBUF
            pltpu.async_copy(h_hbm.at[sidx.at[j + LEAD]], rows.at[bg], gsem.at[bg])
            pltpu.make_async_copy(h_hbm.at[sidx.at[j]], rows.at[b], gsem.at[b]).wait()
            pltpu.async_copy(rows.at[b], acc.at[didx.at[j]], ssem.at[b], add=True)
        return carry

    lax.fori_loop(0, NGRP, group, 0)

    for b in range(NCH - NGRP * NBUF):
        j = NGRP * NBUF + b
        bw = (b + NBUF - LEAD) % NBUF
        pltpu.make_async_copy(rows.at[bw], acc.at[didx.at[j - LEAD]], ssem.at[bw]).wait()
        pltpu.make_async_copy(h_hbm.at[sidx.at[j]], rows.at[b], gsem.at[b]).wait()
        pltpu.async_copy(rows.at[b], acc.at[didx.at[j]], ssem.at[b], add=True)
    for b in range(NCH - NGRP * NBUF):
        j = NGRP * NBUF + b
        pltpu.make_async_copy(rows.at[b], acc.at[didx.at[j]], ssem.at[b]).wait()

    plsc.subcore_barrier()
    pltpu.sync_copy(acc.at[pl.ds(r0, EV)], out_hbm.at[c, pl.ds(r0, EV)])

    @pl.when(s == NS - 1)
    def _evac_tail():
        pltpu.sync_copy(acc.at[pl.ds(N - TAIL, TAIL)], out_hbm.at[c, pl.ds(N - TAIL, TAIL)])


RB = 5000
_GRID = N // RB


def _mm1_body(x_ref, w_ref, o_ref):
    o_ref[...] = jnp.dot(x_ref[...], w_ref[...], preferred_element_type=jnp.float32)


def _norm1_body(do_ref, di_ref, p_ref, h_ref, doi_ref, dii_ref):
    deg_o = do_ref[0] + do_ref[1]
    deg_i = di_ref[0] + di_ref[1]
    doi = lax.rsqrt(jnp.maximum(deg_o, 1.0))
    dii = lax.rsqrt(jnp.maximum(deg_i, 1.0))
    doi_ref[...] = doi
    dii_ref[...] = dii
    h_ref[...] = p_ref[...] * doi[:, 0:1]


def _layer2_body(a_ref, dii_ref, doi_ref, b1_ref, w2_ref, o_ref):
    agg = a_ref[0] + a_ref[1]
    h1 = jnp.maximum(agg * dii_ref[...][:, 0:1] + b1_ref[...], 0.0)
    o_ref[...] = jnp.dot(h1, w2_ref[...],
                         preferred_element_type=jnp.float32) * doi_ref[...][:, 0:1]


def _head_body(a_ref, dii_ref, b2_ref, mw1_ref, mb1_ref, mw2_ref, mb2_ref, o_ref):
    agg = a_ref[0] + a_ref[1]
    h2 = jnp.maximum(agg * dii_ref[...][:, 0:1] + b2_ref[...], 0.0)
    m = jnp.maximum(
        jnp.dot(h2, mw1_ref[...], preferred_element_type=jnp.float32) + mb1_ref[...],
        0.0)
    o_ref[...] = jnp.dot(m, mw2_ref[...],
                         preferred_element_type=jnp.float32) + mb2_ref[...]


@jax.jit
def kernel(edge_index, x, W1, b1, W2, b2, mW1, mb1, mW2, mb2):
    ei = edge_index.reshape(2, NW, NCH, CH)
    ones16 = jnp.ones((CH, 16), jnp.float32)
    z16 = jnp.zeros((EV, 16), jnp.float32)
    z32 = jnp.zeros((EV, H), jnp.float32)

    dout_p, din_p = _deg_kernel(ei, ones16, z16)

    p1 = pl.pallas_call(
        _mm1_body,
        grid=(_GRID,),
        in_specs=[
            pl.BlockSpec((RB, D), lambda i: (i, 0)),
            pl.BlockSpec((D, H), lambda i: (0, 0)),
        ],
        out_specs=pl.BlockSpec((RB, H), lambda i: (i, 0)),
        out_shape=jax.ShapeDtypeStruct((N, H), jnp.float32),
    )(x, W1)

    h1s, doi, dii = pl.pallas_call(
        _norm1_body,
        grid=(_GRID,),
        in_specs=[
            pl.BlockSpec((NC, RB, 16), lambda i: (0, i, 0)),
            pl.BlockSpec((NC, RB, 16), lambda i: (0, i, 0)),
            pl.BlockSpec((RB, H), lambda i: (i, 0)),
        ],
        out_specs=[
            pl.BlockSpec((RB, H), lambda i: (i, 0)),
            pl.BlockSpec((RB, 16), lambda i: (i, 0)),
            pl.BlockSpec((RB, 16), lambda i: (i, 0)),
        ],
        out_shape=[
            jax.ShapeDtypeStruct((N, H), jnp.float32),
            jax.ShapeDtypeStruct((N, 16), jnp.float32),
            jax.ShapeDtypeStruct((N, 16), jnp.float32),
        ],
    )(dout_p, din_p, p1)

    a1 = _agg_kernel(h1s, ei, z32)

    h2s = pl.pallas_call(
        _layer2_body,
        grid=(_GRID,),
        in_specs=[
            pl.BlockSpec((NC, RB, H), lambda i: (0, i, 0)),
            pl.BlockSpec((RB, 16), lambda i: (i, 0)),
            pl.BlockSpec((RB, 16), lambda i: (i, 0)),
            pl.BlockSpec((1, H), lambda i: (0, 0)),
            pl.BlockSpec((H, H), lambda i: (0, 0)),
        ],
        out_specs=pl.BlockSpec((RB, H), lambda i: (i, 0)),
        out_shape=jax.ShapeDtypeStruct((N, H), jnp.float32),
    )(a1, dii, doi, b1.reshape(1, H), W2)

    a2 = _agg_kernel(h2s, ei, z32)

    out = pl.pallas_call(
        _head_body,
        grid=(_GRID,),
        in_specs=[
            pl.BlockSpec((NC, RB, H), lambda i: (0, i, 0)),
            pl.BlockSpec((RB, 16), lambda i: (i, 0)),
            pl.BlockSpec((1, H), lambda i: (0, 0)),
            pl.BlockSpec((H, H), lambda i: (0, 0)),
            pl.BlockSpec((1, H), lambda i: (0, 0)),
            pl.BlockSpec((H, C), lambda i: (0, 0)),
            pl.BlockSpec((1, C), lambda i: (0, 0)),
        ],
        out_specs=pl.BlockSpec((RB, C), lambda i: (i, 0)),
        out_shape=jax.ShapeDtypeStruct((N, C), jnp.float32),
    )(a2, dii, b2.reshape(1, H), mW1, mb1.reshape(1, H), mW2, mb2.reshape(1, C))

    return out


# speedup vs baseline: 22.2508x; 1.3049x over previous
import functools

import jax
import jax.numpy as jnp
from jax import lax
from jax.experimental import pallas as pl
from jax.experimental.pallas import tpu as pltpu
from jax.experimental.pallas import tpu_sc as plsc

N = 10000
E = 320000
D = 128
H = 32
C = 2

NC = 2
NS = 16
NW = NC * NS
EW = E // NW
CH = 80
NCH = EW // CH
NBUF = 10
LEAD = 5
NGRP = (NCH - LEAD) // NBUF
EV = 624
TAIL = N - EV * NS

_mesh = plsc.VectorSubcoreMesh(core_axis_name="c", subcore_axis_name="s")


@functools.partial(
    pl.kernel,
    out_type=(
        jax.ShapeDtypeStruct((NC, N, 16), jnp.float32),
        jax.ShapeDtypeStruct((NC, N, 16), jnp.float32),
    ),
    mesh=_mesh,
    compiler_params=pltpu.CompilerParams(use_tc_tiling_on_sc=False),
    scratch_types=[
        pltpu.VMEM((NCH, CH), jnp.int32),
        pltpu.VMEM((NCH, CH), jnp.int32),
        pltpu.VMEM((CH, 16), jnp.float32),
        pltpu.VMEM_SHARED((N, 16), jnp.float32),
        pltpu.VMEM_SHARED((N, 16), jnp.float32),
        pltpu.SemaphoreType.DMA,
    ],
)
def _deg_kernel(ei_hbm, ones_hbm, z16_hbm,
                dout_hbm, din_hbm, sidx, didx, ones, acc_o, acc_i, sem0):
    c = lax.axis_index("c")
    s = lax.axis_index("s")
    wid = c * NS + s
    pltpu.sync_copy(ei_hbm.at[0, wid], sidx)
    pltpu.sync_copy(ei_hbm.at[1, wid], didx)
    pltpu.sync_copy(ones_hbm, ones)
    r0 = s * EV
    pltpu.sync_copy(z16_hbm, acc_o.at[pl.ds(r0, EV)])
    pltpu.sync_copy(z16_hbm, acc_i.at[pl.ds(r0, EV)])

    @pl.when(s == NS - 1)
    def _zero_tail():
        pltpu.sync_copy(z16_hbm.at[pl.ds(0, TAIL)], acc_o.at[pl.ds(N - TAIL, TAIL)])
        pltpu.sync_copy(z16_hbm.at[pl.ds(0, TAIL)], acc_i.at[pl.ds(N - TAIL, TAIL)])

    plsc.subcore_barrier()

    LAG = 10

    def chunk(j, carry):
        pltpu.async_copy(ones, acc_o.at[sidx.at[j]], sem0, add=True)
        pltpu.async_copy(ones, acc_i.at[didx.at[j]], sem0, add=True)

        @pl.when(j >= LAG)
        def _drain_lagged():
            pltpu.make_async_copy(ones, acc_o.at[sidx.at[j - LAG]], sem0).wait()
            pltpu.make_async_copy(ones, acc_i.at[didx.at[j - LAG]], sem0).wait()

        return carry

    lax.fori_loop(0, NCH, chunk, 0)

    def drain(j, carry):
        pltpu.make_async_copy(ones, acc_o.at[sidx.at[j]], sem0).wait()
        pltpu.make_async_copy(ones, acc_i.at[didx.at[j]], sem0).wait()
        return carry

    lax.fori_loop(NCH - LAG, NCH, drain, 0)
    plsc.subcore_barrier()
    pltpu.sync_copy(acc_o.at[pl.ds(r0, EV)], dout_hbm.at[c, pl.ds(r0, EV)])
    pltpu.sync_copy(acc_i.at[pl.ds(r0, EV)], din_hbm.at[c, pl.ds(r0, EV)])

    @pl.when(s == NS - 1)
    def _evac_tail():
        pltpu.sync_copy(acc_o.at[pl.ds(N - TAIL, TAIL)], dout_hbm.at[c, pl.ds(N - TAIL, TAIL)])
        pltpu.sync_copy(acc_i.at[pl.ds(N - TAIL, TAIL)], din_hbm.at[c, pl.ds(N - TAIL, TAIL)])


@functools.partial(
    pl.kernel,
    out_type=jax.ShapeDtypeStruct((NC, N, H), jnp.float32),
    mesh=_mesh,
    compiler_params=pltpu.CompilerParams(use_tc_tiling_on_sc=False),
    scratch_types=[
        pltpu.VMEM((NCH, CH), jnp.int32),
        pltpu.VMEM((NCH, CH), jnp.int32),
        pltpu.VMEM((NBUF, CH, H), jnp.float32),
        pltpu.VMEM_SHARED((N, H), jnp.float32),
        pltpu.SemaphoreType.DMA((NBUF,)),
        pltpu.SemaphoreType.DMA((NBUF,)),
    ],
)
def _agg_kernel(h_hbm, ei_hbm, z32_hbm, out_hbm,
                sidx, didx, rows, acc, gsem, ssem):
    c = lax.axis_index("c")
    s = lax.axis_index("s")
    wid = c * NS + s
    pltpu.sync_copy(ei_hbm.at[0, wid], sidx)
    pltpu.sync_copy(ei_hbm.at[1, wid], didx)
    for b in range(LEAD):
        pltpu.async_copy(h_hbm.at[sidx.at[b]], rows.at[b], gsem.at[b])
    r0 = s * EV
    pltpu.sync_copy(z32_hbm, acc.at[pl.ds(r0, EV)])

    @pl.when(s == NS - 1)
    def _zero_tail():
        pltpu.sync_copy(z32_hbm.at[pl.ds(0, TAIL)], acc.at[pl.ds(N - TAIL, TAIL)])

    plsc.subcore_barrier()

    def group(g, carry):
        base = g * NBUF
        for b in range(NBUF):
            j = base + b
            bw = (b - LEAD) % NBUF
            if b < LEAD:
                @pl.when(g >= 1)
                def _retire():
                    pltpu.make_async_copy(rows.at[bw], acc.at[didx.at[j - LEAD]],
                                          ssem.at[bw]).wait()
            else:
                pltpu.make_async_copy(rows.at[bw], acc.at[didx.at[j - LEAD]],
                                      ssem.at[bw]).wait()
            bg = (b + LEAD) % NBUF
            pltpu.async_copy(h_hbm.at[sidx.at[j + LEAD]], rows.at[bg], gsem.at[bg])
            pltpu.make_async_copy(h_hbm.at[sidx.at[j]], rows.at[b], gsem.at[b]).wait()
            pltpu.async_copy(rows.at[b], acc.at[didx.at[j]], ssem.at[b], add=True)
        return carry

    lax.fori_loop(0, NGRP, group, 0)

    for b in range(NCH - NGRP * NBUF):
        j = NGRP * NBUF + b
        bw = (b + NBUF - LEAD) % NBUF
        pltpu.make_async_copy(rows.at[bw], acc.at[didx.at[j - LEAD]], ssem.at[bw]).wait()
        pltpu.make_async_copy(h_hbm.at[sidx.at[j]], rows.at[b], gsem.at[b]).wait()
        pltpu.async_copy(rows.at[b], acc.at[didx.at[j]], ssem.at[b], add=True)
    for b in range(NCH - NGRP * NBUF):
        j = NGRP * NBUF + b
        pltpu.make_async_copy(rows.at[b], acc.at[didx.at[j]], ssem.at[b]).wait()

    plsc.subcore_barrier()
    pltpu.sync_copy(acc.at[pl.ds(r0, EV)], out_hbm.at[c, pl.ds(r0, EV)])

    @pl.when(s == NS - 1)
    def _evac_tail():
        pltpu.sync_copy(acc.at[pl.ds(N - TAIL, TAIL)], out_hbm.at[c, pl.ds(N - TAIL, TAIL)])


RB = 5000
_GRID = N // RB


def _mm1_body(x_ref, w_ref, o_ref):
    o_ref[...] = jnp.dot(x_ref[...], w_ref[...], preferred_element_type=jnp.float32)


NR = N * 16 // 128
NH = N * H // 128
PK = 128 // H


def _blockdiag(w, k):
    r, c = w.shape
    out = jnp.zeros((k * r, k * c), jnp.float32)
    for i in range(k):
        out = out.at[i * r:(i + 1) * r, i * c:(i + 1) * c].set(w)
    return out


def _expand_scale(v128):
    d2 = jnp.broadcast_to(v128[:, None, :], (NR, 2, 128)).reshape(NH, 128)
    li = lax.broadcasted_iota(jnp.int32, (128, 128), 0)
    ci = lax.broadcasted_iota(jnp.int32, (128, 128), 1)
    p0 = (li == 16 * (ci // H)).astype(jnp.float32)
    p1 = (li == 16 * (PK + ci // H)).astype(jnp.float32)
    u_e = jnp.dot(d2, p0, preferred_element_type=jnp.float32)
    u_o = jnp.dot(d2, p1, preferred_element_type=jnp.float32)
    parity = lax.broadcasted_iota(jnp.int32, (NH, 1), 0) % 2 == 0
    return jnp.where(parity, u_e, u_o)


def _mm1_body(x_ref, w_ref, o_ref):
    o_ref[...] = jnp.dot(x_ref[...], w_ref[...], preferred_element_type=jnp.float32)


def _norm1_body(do_ref, di_ref, p_ref, h_ref, doi_ref, dii_ref):
    doi = lax.rsqrt(jnp.maximum(do_ref[0] + do_ref[1], 1.0))
    dii = lax.rsqrt(jnp.maximum(di_ref[0] + di_ref[1], 1.0))
    doi32 = _expand_scale(doi)
    dii32 = _expand_scale(dii)
    doi_ref[...] = doi32
    dii_ref[...] = dii32
    h_ref[...] = p_ref[...] * doi32


def _layer2_body(a_ref, dii_ref, doi_ref, b1_ref, w2_ref, o_ref):
    agg = a_ref[0] + a_ref[1]
    h1 = jnp.maximum(agg * dii_ref[...] + b1_ref[...], 0.0)
    o_ref[...] = jnp.dot(h1, w2_ref[...],
                         preferred_element_type=jnp.float32) * doi_ref[...]


def _head_body(a_ref, dii_ref, b2_ref, mw1_ref, mb1_ref, mw2_ref, mb2_ref, o_ref):
    agg = a_ref[0] + a_ref[1]
    h2 = jnp.maximum(agg * dii_ref[...] + b2_ref[...], 0.0)
    m = jnp.maximum(
        jnp.dot(h2, mw1_ref[...], preferred_element_type=jnp.float32) + mb1_ref[...],
        0.0)
    o_ref[...] = jnp.dot(m, mw2_ref[...],
                         preferred_element_type=jnp.float32) + mb2_ref[...]


def _full(shape):
    nd = len(shape)
    return pl.BlockSpec(shape, lambda: (0,) * nd)


@jax.jit
def kernel(edge_index, x, W1, b1, W2, b2, mW1, mb1, mW2, mb2):
    ei = edge_index.reshape(2, NW, NCH, CH)
    ones16 = jnp.ones((CH, 16), jnp.float32)
    z16 = jnp.zeros((EV, 16), jnp.float32)
    z32 = jnp.zeros((EV, H), jnp.float32)
    x4 = x.reshape(NH, PK * D)
    w1bd = _blockdiag(W1, PK)
    w2bd = _blockdiag(W2, PK)
    mw1bd = _blockdiag(mW1, PK)
    mw2s = _blockdiag(mW2, PK)
    b1t = jnp.tile(b1, PK).reshape(1, 128)
    b2t = jnp.tile(b2, PK).reshape(1, 128)
    mb1t = jnp.tile(mb1, PK).reshape(1, 128)
    mb2t = jnp.tile(mb2, PK).reshape(1, PK * C)

    dout_p, din_p = _deg_kernel(ei, ones16, z16)
    dout128 = dout_p.reshape(NC, NR, 128)
    din128 = din_p.reshape(NC, NR, 128)

    p1 = pl.pallas_call(
        _mm1_body,
        in_specs=[_full((NH, PK * D)), _full((PK * D, 128))],
        out_specs=_full((NH, 128)),
        out_shape=jax.ShapeDtypeStruct((NH, 128), jnp.float32),
    )(x4, w1bd)

    h1s, doi32, dii32 = pl.pallas_call(
        _norm1_body,
        in_specs=[_full((NC, NR, 128)), _full((NC, NR, 128)), _full((NH, 128))],
        out_specs=[_full((NH, 128)), _full((NH, 128)), _full((NH, 128))],
        out_shape=[
            jax.ShapeDtypeStruct((NH, 128), jnp.float32),
            jax.ShapeDtypeStruct((NH, 128), jnp.float32),
            jax.ShapeDtypeStruct((NH, 128), jnp.float32),
        ],
    )(dout128, din128, p1)

    a1 = _agg_kernel(h1s.reshape(N, H), ei, z32)

    h2s = pl.pallas_call(
        _layer2_body,
        in_specs=[_full((NC, NH, 128)), _full((NH, 128)), _full((NH, 128)),
                  _full((1, 128)), _full((128, 128))],
        out_specs=_full((NH, 128)),
        out_shape=jax.ShapeDtypeStruct((NH, 128), jnp.float32),
    )(a1.reshape(NC, NH, 128), dii32, doi32, b1t, w2bd)

    a2 = _agg_kernel(h2s.reshape(N, H), ei, z32)

    out8 = pl.pallas_call(
        _head_body,
        in_specs=[_full((NC, NH, 128)), _full((NH, 128)), _full((1, 128)),
                  _full((128, 128)), _full((1, 128)), _full((128, PK * C)),
                  _full((1, PK * C))],
        out_specs=_full((NH, PK * C)),
        out_shape=jax.ShapeDtypeStruct((NH, PK * C), jnp.float32),
    )(a2.reshape(NC, NH, 128), dii32, b2t, mw1bd, mb1t, mw2s, mb2t)

    return out8.reshape(N, C)


# speedup vs baseline: 22.2994x; 1.0022x over previous
import functools

import jax
import jax.numpy as jnp
from jax import lax
from jax.experimental import pallas as pl
from jax.experimental.pallas import tpu as pltpu
from jax.experimental.pallas import tpu_sc as plsc

N = 10000
E = 320000
D = 128
H = 32
C = 2

NC = 2
NS = 16
NW = NC * NS
EW = E // NW
CH = 80
NCH = EW // CH
NBUF = 10
LEAD = 5
NGRP = (NCH - LEAD) // NBUF
EV = 624
TAIL = N - EV * NS

_mesh = plsc.VectorSubcoreMesh(core_axis_name="c", subcore_axis_name="s")


@functools.partial(
    pl.kernel,
    out_type=(
        jax.ShapeDtypeStruct((NC, N, 16), jnp.float32),
        jax.ShapeDtypeStruct((NC, N, 16), jnp.float32),
    ),
    mesh=_mesh,
    compiler_params=pltpu.CompilerParams(use_tc_tiling_on_sc=False),
    scratch_types=[
        pltpu.VMEM((NCH, CH), jnp.int32),
        pltpu.VMEM((NCH, CH), jnp.int32),
        pltpu.VMEM((CH, 16), jnp.float32),
        pltpu.VMEM_SHARED((N, 16), jnp.float32),
        pltpu.VMEM_SHARED((N, 16), jnp.float32),
        pltpu.SemaphoreType.DMA,
    ],
)
def _deg_kernel(ei_hbm, ones_hbm, z16_hbm,
                dout_hbm, din_hbm, sidx, didx, ones, acc_o, acc_i, sem0):
    c = lax.axis_index("c")
    s = lax.axis_index("s")
    wid = c * NS + s
    pltpu.sync_copy(ei_hbm.at[0, wid], sidx)
    pltpu.sync_copy(ei_hbm.at[1, wid], didx)
    pltpu.sync_copy(ones_hbm, ones)
    r0 = s * EV
    pltpu.sync_copy(z16_hbm, acc_o.at[pl.ds(r0, EV)])
    pltpu.sync_copy(z16_hbm, acc_i.at[pl.ds(r0, EV)])

    @pl.when(s == NS - 1)
    def _zero_tail():
        pltpu.sync_copy(z16_hbm.at[pl.ds(0, TAIL)], acc_o.at[pl.ds(N - TAIL, TAIL)])
        pltpu.sync_copy(z16_hbm.at[pl.ds(0, TAIL)], acc_i.at[pl.ds(N - TAIL, TAIL)])

    plsc.subcore_barrier()

    LAG = 30

    def chunk(j, carry):
        pltpu.async_copy(ones, acc_o.at[sidx.at[j]], sem0, add=True)
        pltpu.async_copy(ones, acc_i.at[didx.at[j]], sem0, add=True)

        @pl.when(j >= LAG)
        def _drain_lagged():
            pltpu.make_async_copy(ones, acc_o.at[sidx.at[j - LAG]], sem0).wait()
            pltpu.make_async_copy(ones, acc_i.at[didx.at[j - LAG]], sem0).wait()

        return carry

    lax.fori_loop(0, NCH, chunk, 0)

    def drain(j, carry):
        pltpu.make_async_copy(ones, acc_o.at[sidx.at[j]], sem0).wait()
        pltpu.make_async_copy(ones, acc_i.at[didx.at[j]], sem0).wait()
        return carry

    lax.fori_loop(NCH - LAG, NCH, drain, 0)
    plsc.subcore_barrier()
    pltpu.sync_copy(acc_o.at[pl.ds(r0, EV)], dout_hbm.at[c, pl.ds(r0, EV)])
    pltpu.sync_copy(acc_i.at[pl.ds(r0, EV)], din_hbm.at[c, pl.ds(r0, EV)])

    @pl.when(s == NS - 1)
    def _evac_tail():
        pltpu.sync_copy(acc_o.at[pl.ds(N - TAIL, TAIL)], dout_hbm.at[c, pl.ds(N - TAIL, TAIL)])
        pltpu.sync_copy(acc_i.at[pl.ds(N - TAIL, TAIL)], din_hbm.at[c, pl.ds(N - TAIL, TAIL)])


@functools.partial(
    pl.kernel,
    out_type=jax.ShapeDtypeStruct((NC, N, H), jnp.float32),
    mesh=_mesh,
    compiler_params=pltpu.CompilerParams(use_tc_tiling_on_sc=False),
    scratch_types=[
        pltpu.VMEM((NCH, CH), jnp.int32),
        pltpu.VMEM((NCH, CH), jnp.int32),
        pltpu.VMEM((NBUF, CH, H), jnp.float32),
        pltpu.VMEM_SHARED((N, H), jnp.float32),
        pltpu.SemaphoreType.DMA((NBUF,)),
        pltpu.SemaphoreType.DMA((NBUF,)),
    ],
)
def _agg_kernel(h_hbm, ei_hbm, z32_hbm, out_hbm,
                sidx, didx, rows, acc, gsem, ssem):
    c = lax.axis_index("c")
    s = lax.axis_index("s")
    wid = c * NS + s
    pltpu.sync_copy(ei_hbm.at[0, wid], sidx)
    pltpu.sync_copy(ei_hbm.at[1, wid], didx)
    for b in range(LEAD):
        pltpu.async_copy(h_hbm.at[sidx.at[b]], rows.at[b], gsem.at[b])
    r0 = s * EV
    pltpu.sync_copy(z32_hbm, acc.at[pl.ds(r0, EV)])

    @pl.when(s == NS - 1)
    def _zero_tail():
        pltpu.sync_copy(z32_hbm.at[pl.ds(0, TAIL)], acc.at[pl.ds(N - TAIL, TAIL)])

    plsc.subcore_barrier()

    def group(g, carry):
        base = g * NBUF
        for b in range(NBUF):
            j = base + b
            bw = (b - LEAD) % NBUF
            if b < LEAD:
                @pl.when(g >= 1)
                def _retire():
                    pltpu.make_async_copy(rows.at[bw], acc.at[didx.at[j - LEAD]],
                                          ssem.at[bw]).wait()
            else:
                pltpu.make_async_copy(rows.at[bw], acc.at[didx.at[j - LEAD]],
                                      ssem.at[bw]).wait()
            bg = (b + LEAD) % NBUF
            pltpu.async_copy(h_hbm.at[sidx.at[j + LEAD]], rows.at[bg], gsem.at[bg])
            pltpu.make_async_copy(h_hbm.at[sidx.at[j]], rows.at[b], gsem.at[b]).wait()
            pltpu.async_copy(rows.at[b], acc.at[didx.at[j]], ssem.at[b], add=True)
        return carry

    lax.fori_loop(0, NGRP, group, 0)

    for b in range(NCH - NGRP * NBUF):
        j = NGRP * NBUF + b
        bw = (b + NBUF - LEAD) % NBUF
        pltpu.make_async_copy(rows.at[bw], acc.at[didx.at[j - LEAD]], ssem.at[bw]).wait()
        pltpu.make_async_copy(h_hbm.at[sidx.at[j]], rows.at[b], gsem.at[b]).wait()
        pltpu.async_copy(rows.at[b], acc.at[didx.at[j]], ssem.at[b], add=True)
    for b in range(NCH - NGRP * NBUF):
        j = NGRP * NBUF + b
        pltpu.make_async_copy(rows.at[b], acc.at[didx.at[j]], ssem.at[b]).wait()

    plsc.subcore_barrier()
    pltpu.sync_copy(acc.at[pl.ds(r0, EV)], out_hbm.at[c, pl.ds(r0, EV)])

    @pl.when(s == NS - 1)
    def _evac_tail():
        pltpu.sync_copy(acc.at[pl.ds(N - TAIL, TAIL)], out_hbm.at[c, pl.ds(N - TAIL, TAIL)])


RB = 5000
_GRID = N // RB


def _mm1_body(x_ref, w_ref, o_ref):
    o_ref[...] = jnp.dot(x_ref[...], w_ref[...], preferred_element_type=jnp.float32)


NR = N * 16 // 128
NH = N * H // 128
PK = 128 // H


def _blockdiag(w, k):
    r, c = w.shape
    out = jnp.zeros((k * r, k * c), jnp.float32)
    for i in range(k):
        out = out.at[i * r:(i + 1) * r, i * c:(i + 1) * c].set(w)
    return out


def _expand_scale(v128):
    d2 = jnp.broadcast_to(v128[:, None, :], (NR, 2, 128)).reshape(NH, 128)
    li = lax.broadcasted_iota(jnp.int32, (128, 128), 0)
    ci = lax.broadcasted_iota(jnp.int32, (128, 128), 1)
    p0 = (li == 16 * (ci // H)).astype(jnp.float32)
    p1 = (li == 16 * (PK + ci // H)).astype(jnp.float32)
    u_e = jnp.dot(d2, p0, preferred_element_type=jnp.float32)
    u_o = jnp.dot(d2, p1, preferred_element_type=jnp.float32)
    parity = lax.broadcasted_iota(jnp.int32, (NH, 1), 0) % 2 == 0
    return jnp.where(parity, u_e, u_o)


def _mm1_body(x_ref, w_ref, o_ref):
    o_ref[...] = jnp.dot(x_ref[...], w_ref[...], preferred_element_type=jnp.float32)


def _norm1_body(do_ref, di_ref, p_ref, h_ref, doi_ref, dii_ref):
    doi = lax.rsqrt(jnp.maximum(do_ref[0] + do_ref[1], 1.0))
    dii = lax.rsqrt(jnp.maximum(di_ref[0] + di_ref[1], 1.0))
    doi32 = _expand_scale(doi)
    dii32 = _expand_scale(dii)
    doi_ref[...] = doi32
    dii_ref[...] = dii32
    h_ref[...] = p_ref[...] * doi32


def _layer2_body(a_ref, dii_ref, doi_ref, b1_ref, w2_ref, o_ref):
    agg = a_ref[0] + a_ref[1]
    h1 = jnp.maximum(agg * dii_ref[...] + b1_ref[...], 0.0)
    o_ref[...] = jnp.dot(h1, w2_ref[...],
                         preferred_element_type=jnp.float32) * doi_ref[...]


def _head_body(a_ref, dii_ref, b2_ref, mw1_ref, mb1_ref, mw2_ref, mb2_ref, o_ref):
    agg = a_ref[0] + a_ref[1]
    h2 = jnp.maximum(agg * dii_ref[...] + b2_ref[...], 0.0)
    m = jnp.maximum(
        jnp.dot(h2, mw1_ref[...], preferred_element_type=jnp.float32) + mb1_ref[...],
        0.0)
    o_ref[...] = jnp.dot(m, mw2_ref[...],
                         preferred_element_type=jnp.float32) + mb2_ref[...]


def _full(shape):
    nd = len(shape)
    return pl.BlockSpec(shape, lambda: (0,) * nd)


@jax.jit
def kernel(edge_index, x, W1, b1, W2, b2, mW1, mb1, mW2, mb2):
    ei = edge_index.reshape(2, NW, NCH, CH)
    ones16 = jnp.ones((CH, 16), jnp.float32)
    z16 = jnp.zeros((EV, 16), jnp.float32)
    z32 = jnp.zeros((EV, H), jnp.float32)
    x4 = x.reshape(NH, PK * D)
    w1bd = _blockdiag(W1, PK)
    w2bd = _blockdiag(W2, PK)
    mw1bd = _blockdiag(mW1, PK)
    mw2s = _blockdiag(mW2, PK)
    b1t = jnp.tile(b1, PK).reshape(1, 128)
    b2t = jnp.tile(b2, PK).reshape(1, 128)
    mb1t = jnp.tile(mb1, PK).reshape(1, 128)
    mb2t = jnp.tile(mb2, PK).reshape(1, PK * C)

    dout_p, din_p = _deg_kernel(ei, ones16, z16)
    dout128 = dout_p.reshape(NC, NR, 128)
    din128 = din_p.reshape(NC, NR, 128)

    p1 = pl.pallas_call(
        _mm1_body,
        in_specs=[_full((NH, PK * D)), _full((PK * D, 128))],
        out_specs=_full((NH, 128)),
        out_shape=jax.ShapeDtypeStruct((NH, 128), jnp.float32),
    )(x4, w1bd)

    h1s, doi32, dii32 = pl.pallas_call(
        _norm1_body,
        in_specs=[_full((NC, NR, 128)), _full((NC, NR, 128)), _full((NH, 128))],
        out_specs=[_full((NH, 128)), _full((NH, 128)), _full((NH, 128))],
        out_shape=[
            jax.ShapeDtypeStruct((NH, 128), jnp.float32),
            jax.ShapeDtypeStruct((NH, 128), jnp.float32),
            jax.ShapeDtypeStruct((NH, 128), jnp.float32),
        ],
    )(dout128, din128, p1)

    a1 = _agg_kernel(h1s.reshape(N, H), ei, z32)

    h2s = pl.pallas_call(
        _layer2_body,
        in_specs=[_full((NC, NH, 128)), _full((NH, 128)), _full((NH, 128)),
                  _full((1, 128)), _full((128, 128))],
        out_specs=_full((NH, 128)),
        out_shape=jax.ShapeDtypeStruct((NH, 128), jnp.float32),
    )(a1.reshape(NC, NH, 128), dii32, doi32, b1t, w2bd)

    a2 = _agg_kernel(h2s.reshape(N, H), ei, z32)

    out8 = pl.pallas_call(
        _head_body,
        in_specs=[_full((NC, NH, 128)), _full((NH, 128)), _full((1, 128)),
                  _full((128, 128)), _full((1, 128)), _full((128, PK * C)),
                  _full((1, PK * C))],
        out_specs=_full((NH, PK * C)),
        out_shape=jax.ShapeDtypeStruct((NH, PK * C), jnp.float32),
    )(a2.reshape(NC, NH, 128), dii32, b2t, mw1bd, mb1t, mw2s, mb2t)

    return out8.reshape(N, C)
